# baseline (speedup 1.0000x reference)
import functools

import jax
import jax.numpy as jnp
from jax import lax
from jax.experimental import pallas as pl
from jax.experimental.pallas import tpu as pltpu
from jax.experimental.pallas import tpu_sc as plsc

LANES = 16
NBUF = 2


@functools.lru_cache(maxsize=None)
def _make_emb_kernel(B, S, D, V):
    info = plsc.get_sparse_core_info()
    NC, NS = info.num_cores, info.num_subcores
    NW = NC * NS
    BB = B // 128
    nq = S * BB
    assert B % 128 == 0 and D % 8 == 0 and nq % (NW * NBUF) == 0, (B, S, D)
    DG = D // 8
    niter = nq // (NW * NBUF)
    mesh = plsc.VectorSubcoreMesh(core_axis_name="c", subcore_axis_name="s")

    @functools.partial(
        pl.kernel,
        mesh=mesh,
        compiler_params=pltpu.CompilerParams(
            use_tc_tiling_on_sc=False, needs_layout_passes=False),
        out_type=jax.ShapeDtypeStruct((S, DG, BB, 8, 128), jnp.float32),
        scratch_types=[
            pltpu.VMEM((S, D), jnp.float32),
            [pltpu.VMEM((128,), jnp.int32)] * NBUF,
            [pltpu.VMEM((128, D), jnp.float32)] * NBUF,
            [pltpu.VMEM((DG, 8, 128), jnp.float32)] * NBUF,
            [pltpu.SemaphoreType.DMA] * NBUF,
            [pltpu.SemaphoreType.DMA] * NBUF,
            [pltpu.SemaphoreType.DMA] * NBUF,
        ],
    )
    def emb_kernel(idxf_hbm, wte_hbm, wpe_hbm, out_hbm,
                   wpe_v, idx_v, rows_v, t_v, i_sems, g_sems, o_sems):
        wid = lax.axis_index("s") * NC + lax.axis_index("c")
        pltpu.sync_copy(wpe_hbm, wpe_v)
        iota = lax.iota(jnp.int32, LANES)

        def start_idx(q, b):
            s = q // BB
            bg = q - s * BB
            return pltpu.async_copy(
                idxf_hbm.at[pl.ds(s * B + bg * 128, 128)], idx_v[b], i_sems[b])

        def wait_idx(b):
            pltpu.make_async_copy(
                idxf_hbm.at[pl.ds(0, 128)], idx_v[b], i_sems[b]).wait()

        def start_gather(b):
            return pltpu.async_copy(
                wte_hbm.at[idx_v[b]], rows_v[b], g_sems[b])

        def wait_gather(b):
            pltpu.make_async_copy(
                wte_hbm.at[idx_v[b]], rows_v[b], g_sems[b]).wait()

        def wait_out(b):
            pltpu.make_async_copy(
                t_v[b], out_hbm.at[0, :, 0], o_sems[b]).wait()

        for b in range(NBUF):
            start_idx(wid * NBUF + b, b)
        for b in range(NBUF):
            wait_idx(b)
            start_gather(b)

        def iter_body(i, _):
            q0 = i * NW * NBUF + wid * NBUF
            for b in range(NBUF):
                q = q0 + b
                s = q // BB
                bg = q - s * BB
                wait_gather(b)

                @pl.when(i + 1 < niter)
                def _():
                    start_idx(q + NW * NBUF, b)

                @pl.when(i > 0)
                def _():
                    wait_out(b)

                svec = jnp.full((LANES,), s, jnp.int32)

                @plsc.parallel_loop(0, D, 1, unroll=2)
                def _(d):
                    dvec = jnp.full((LANES,), d, jnp.int32)
                    wsd = plsc.load_gather(wpe_v, [svec, dvec])
                    dg = d // 8
                    dr = d - dg * 8
                    for g in range(128 // LANES):
                        rvec = iota + (g * LANES)
                        val = plsc.load_gather(rows_v[b], [rvec, dvec])
                        t_v[b][dg, dr, pl.ds(g * LANES, LANES)] = val + wsd

                pltpu.async_copy(t_v[b], out_hbm.at[s, :, bg], o_sems[b])

                @pl.when(i + 1 < niter)
                def _():
                    wait_idx(b)
                    start_gather(b)

            return 0

        lax.fori_loop(0, niter, iter_body, 0)
        for b in range(NBUF):
            wait_out(b)

    return emb_kernel


def kernel(idx, wte, wpe):
    B, S = idx.shape
    V, D = wte.shape
    out4 = _make_emb_kernel(B, S, D, V)(idx.T.reshape(-1), wte, wpe)
    return out4.transpose(2, 4, 0, 1, 3).reshape(B, S, D)
